# baseline (speedup 1.0000x reference)
import dataclasses
import functools

import jax
import jax.numpy as jnp
from jax import lax
from jax.experimental import pallas as pl
from jax.experimental.pallas import tpu as pltpu
from jax.experimental.pallas import tpu_sc as plsc

B, L, V, E, H = 1024, 50, 1000000, 64, 64
G4 = 4 * H

NC, NS = 2, 16
NW = NC * NS
NIDX = B * L
PER_W = NIDX // NW
CH = 64
NCH = PER_W // CH
VG = V // 8


B_CH = 8
B_PER_C = B // NC


def _sc_gather(emb, x):
    mesh = plsc.ScalarSubcoreMesh(axis_name="core", num_cores=NC)

    @functools.partial(
        pl.kernel,
        mesh=mesh,
        out_type=jax.ShapeDtypeStruct((NIDX, E), jnp.float32),
        scratch_types=[
            pltpu.SMEM((B_CH, L), jnp.int32),
            pltpu.SemaphoreType.DMA,
            pltpu.SemaphoreType.DMA,
        ],
    )
    def gather_kernel(table_hbm, x_hbm, out_hbm, idx_s, isem, osem):
        cid = lax.axis_index("core")
        b0 = cid * B_PER_C

        @pl.loop(0, B_PER_C // B_CH)
        def _(kc):
            bb = b0 + kc * B_CH
            pltpu.async_copy(x_hbm.at[pl.ds(bb, B_CH)], idx_s, isem).wait()

            @pl.loop(0, B_CH)
            def _(k):
                @pl.loop(0, L)
                def _(t):
                    pltpu.make_async_copy(
                        table_hbm.at[idx_s[k, t]],
                        out_hbm.at[t * B + bb + k],
                        osem,
                    ).start()

        pltpu.make_async_copy(
            table_hbm.at[pl.ds(0, B_PER_C * L)],
            out_hbm.at[pl.ds(b0 * L, B_PER_C * L)],
            osem,
        ).wait()

    return gather_kernel(emb, x)


def _lstm_body(e_ref, wih_ref, whh_ref, b_ref, out_hbm,
               h_ref, c_ref, h_buf, out_sem):
    t = pl.program_id(0)

    @pl.when(t == 0)
    def _():
        h_ref[...] = jnp.zeros((B, H), jnp.float32)
        c_ref[...] = jnp.zeros((B, H), jnp.float32)

    h = h_ref[...]
    c = c_ref[...]
    et = e_ref[...]

    def gate(g):
        w_i = wih_ref[pl.ds(g * E, E), :]
        w_h = whh_ref[pl.ds(g * H, H), :]
        acc = jnp.dot(et, w_i, preferred_element_type=jnp.float32)
        acc += jnp.dot(h, w_h, preferred_element_type=jnp.float32)
        return acc + b_ref[g, :]

    i = jax.nn.sigmoid(gate(0))
    f = jax.nn.sigmoid(gate(1))
    g = jnp.tanh(gate(2))
    o = jax.nn.sigmoid(gate(3))
    c = f * c + i * g
    h = o * jnp.tanh(c)
    h_ref[...] = h
    c_ref[...] = c

    def out_copy(tt, slot):
        return pltpu.make_async_copy(
            h_buf.at[slot], out_hbm.at[:, tt], out_sem.at[slot]
        )

    slot = lax.rem(t, 2)

    @pl.when(t >= 2)
    def _():
        out_copy(t - 2, slot).wait()

    h_buf[slot] = h
    out_copy(t, slot).start()

    @pl.when(t == L - 1)
    def _():
        out_copy(t - 1, lax.rem(t - 1, 2)).wait()
        out_copy(t, slot).wait()


def _lstm_tc(e_flat, wih_s, whh_s, bias4):
    return pl.pallas_call(
        _lstm_body,
        grid=(L,),
        in_specs=[
            pl.BlockSpec((B, E), lambda t: (t, 0)),
            pl.BlockSpec((G4, H), lambda t: (0, 0)),
            pl.BlockSpec((G4, H), lambda t: (0, 0)),
            pl.BlockSpec((4, H), lambda t: (0, 0)),
        ],
        out_specs=pl.BlockSpec(memory_space=pl.ANY),
        out_shape=jax.ShapeDtypeStruct((B, L, H), jnp.float32),
        scratch_shapes=[
            pltpu.VMEM((B, H), jnp.float32),
            pltpu.VMEM((B, H), jnp.float32),
            pltpu.VMEM((2, B, H), jnp.float32),
            pltpu.SemaphoreType.DMA((2,)),
        ],
    )(e_flat, wih_s, whh_s, bias4)


def kernel(x, emb, W_ih, W_hh, b_ih, b_hh):
    e = _sc_gather(emb, x)
    wih_s = W_ih.reshape(4, H, E).transpose(0, 2, 1).reshape(4 * E, H)
    whh_s = W_hh.reshape(4, H, H).transpose(0, 2, 1).reshape(4 * H, H)
    bias4 = (b_ih + b_hh).reshape(4, H)
    return _lstm_tc(e, wih_s, whh_s, bias4)


# speedup vs baseline: 3.8075x; 3.8075x over previous
import dataclasses
import functools

import jax
import jax.numpy as jnp
from jax import lax
from jax.experimental import pallas as pl
from jax.experimental.pallas import tpu as pltpu
from jax.experimental.pallas import tpu_sc as plsc

B, L, V, E, H = 1024, 50, 1000000, 64, 64
G4 = 4 * H


def _lstm_body(e_ref, wih_ref, whh_ref, b_ref, out_ref, h_ref, c_ref):
    t = pl.program_id(0)

    @pl.when(t == 0)
    def _():
        h_ref[...] = jnp.zeros((H, B), jnp.float32)
        c_ref[...] = jnp.zeros((H, B), jnp.float32)

    h = h_ref[...]
    c = c_ref[...]
    et = e_ref[...]

    def gate(g):
        w_i = wih_ref[pl.ds(g * H, H), :]
        w_h = whh_ref[pl.ds(g * H, H), :]
        acc = jnp.dot(w_i, et, preferred_element_type=jnp.float32)
        acc += jnp.dot(w_h, h, preferred_element_type=jnp.float32)
        return acc + b_ref[pl.ds(g * H, H), :]

    i = jax.nn.sigmoid(gate(0))
    f = jax.nn.sigmoid(gate(1))
    g = jnp.tanh(gate(2))
    o = jax.nn.sigmoid(gate(3))
    c = f * c + i * g
    h = o * jnp.tanh(c)
    h_ref[...] = h
    c_ref[...] = c
    out_ref[...] = h.reshape(1, H, B)


def _lstm_tc(e_t, wih, whh, bias2):
    return pl.pallas_call(
        _lstm_body,
        grid=(L,),
        in_specs=[
            pl.BlockSpec((E, B), lambda t: (0, t)),
            pl.BlockSpec((G4, E), lambda t: (0, 0)),
            pl.BlockSpec((G4, H), lambda t: (0, 0)),
            pl.BlockSpec((G4, 1), lambda t: (0, 0)),
        ],
        out_specs=pl.BlockSpec((1, H, B), lambda t: (t, 0, 0)),
        out_shape=jax.ShapeDtypeStruct((L, H, B), jnp.float32),
        scratch_shapes=[
            pltpu.VMEM((H, B), jnp.float32),
            pltpu.VMEM((H, B), jnp.float32),
        ],
    )(e_t, wih, whh, bias2)


def kernel(x, emb, W_ih, W_hh, b_ih, b_hh):
    eT = jnp.take(emb.T, x.T.reshape(-1), axis=1)
    bias2 = (b_ih + b_hh).reshape(G4, 1)
    o = _lstm_tc(eT, W_ih, W_hh, bias2)
    return o.transpose(2, 0, 1)
